# baseline (speedup 1.0000x reference)
import functools

import jax
import jax.numpy as jnp
from jax import lax
from jax.experimental import pallas as pl
from jax.experimental.pallas import tpu as pltpu
from jax.experimental.pallas import tpu_sc as plsc

N = 10000
E = 320000
D = 128
G = 16

NC = 2
NS = 16
NW = NC * NS
EPW = E // NW
CHUNK = 80
NCHUNK = EPW // CHUNK
NGROUP = CHUNK // 16
NBUF = 3
NQ = 2 * NBUF
NSUPER = NCHUNK // NBUF
NTAIL = NCHUNK - NSUPER * NBUF

ROWS_A = 624
ROWS_LAST = N - 15 * ROWS_A

BLK = 2000
GRID = N // BLK


def _sc_body(h_hbm, a_hbm, b_hbm, src_hbm, dst_hbm, znd_hbm, zn_hbm,
             part_hbm, dpart_hbm,
             src_v, dst_v, rows_v, av_v, bv_v, alpha_v,
             accum, denom, stage_sem, isem, gsem, ssem):
    cid = lax.axis_index("c")
    sid = lax.axis_index("s")
    wid = cid * NS + sid
    base_e = wid * EPW

    row0 = pl.multiple_of(sid * ROWS_A, 8)

    @pl.when(sid < NS - 1)
    def _():
        pltpu.async_copy(znd_hbm.at[pl.ds(row0, ROWS_A)],
                         accum.at[pl.ds(row0, ROWS_A)], stage_sem).wait()

    @pl.when(sid == NS - 1)
    def _():
        pltpu.async_copy(znd_hbm.at[pl.ds(15 * ROWS_A, ROWS_LAST)],
                         accum.at[pl.ds(15 * ROWS_A, ROWS_LAST)],
                         stage_sem).wait()

    @pl.when(sid == 0)
    def _():
        pltpu.async_copy(zn_hbm, denom, stage_sem).wait()

    plsc.subcore_barrier()

    def fire_idx(c, q):
        cb = pl.multiple_of(base_e + c * CHUNK, 8)
        pltpu.async_copy(src_hbm.at[pl.ds(cb, CHUNK)], src_v[q], isem[q])
        pltpu.async_copy(dst_hbm.at[pl.ds(cb, CHUNK)], dst_v[q], isem[q])

    def wait_idx(q):
        pltpu.make_async_copy(src_hbm.at[pl.ds(0, CHUNK)], src_v[q],
                              isem[q]).wait()
        pltpu.make_async_copy(dst_hbm.at[pl.ds(0, CHUNK)], dst_v[q],
                              isem[q]).wait()

    def fire_gather(b, q):
        pltpu.async_copy(h_hbm.at[src_v[q]], rows_v[b], gsem[b])
        pltpu.async_copy(a_hbm.at[dst_v[q]], av_v[b], gsem[b])
        pltpu.async_copy(b_hbm.at[src_v[q]], bv_v[b], gsem[b])

    def wait_gather(b):
        pltpu.make_async_copy(znd_hbm.at[pl.ds(0, CHUNK)], rows_v[b],
                              gsem[b]).wait()
        pltpu.make_async_copy(zn_hbm.at[pl.ds(0, CHUNK)], av_v[b],
                              gsem[b]).wait()
        pltpu.make_async_copy(zn_hbm.at[pl.ds(0, CHUNK)], bv_v[b],
                              gsem[b]).wait()

    def drain_scatter(b):
        pltpu.make_async_copy(znd_hbm.at[pl.ds(0, CHUNK)], rows_v[b],
                              ssem[b]).wait()
        pltpu.make_async_copy(zn_hbm.at[pl.ds(0, CHUNK)], alpha_v[b],
                              ssem[b]).wait()

    def compute_chunk(b):
        def group_body(g, carry2):
            e0 = g * 16
            av = av_v[b][pl.ds(e0, 16)]
            bv = bv_v[b][pl.ds(e0, 16)]
            alpha = 1.0 / (1.0 + jnp.exp(-(av + bv)))
            alpha_v[b][pl.ds(e0, 16)] = alpha
            for k in range(16):
                e = e0 + k
                asp = plsc.load_gather(
                    alpha_v[b],
                    [jnp.broadcast_to(e, (16,)).astype(jnp.int32)])
                for r in range(D // 16):
                    sl = pl.ds(r * 16, 16)
                    rows_v[b][e, sl] = rows_v[b][e, sl] * asp
            return carry2

        lax.fori_loop(0, NGROUP, group_body, 0, unroll=False)

    def fire_scatter(b, q):
        pltpu.async_copy(rows_v[b], accum.at[dst_v[q]], ssem[b], add=True)
        pltpu.async_copy(alpha_v[b], denom.at[dst_v[q]], ssem[b], add=True)

    for c in range(NBUF):
        fire_idx(c, c)
    for b in range(NBUF):
        pltpu.async_copy(znd_hbm.at[pl.ds(0, CHUNK)], rows_v[b], ssem[b])
        pltpu.async_copy(zn_hbm.at[pl.ds(0, CHUNK)], alpha_v[b], ssem[b])
    wait_idx(0)
    drain_scatter(0)
    fire_gather(0, 0)

    def one_super(j, qs, qn):
        c0 = j * NBUF
        wait_idx(qs[1])
        drain_scatter(1)
        fire_gather(1, qs[1])
        wait_idx(qs[2])
        drain_scatter(2)
        fire_gather(2, qs[2])
        fire_idx(c0 + 3, qn[0])
        fire_idx(c0 + 4, qn[1])

        @pl.when(c0 + 5 < NCHUNK)
        def _():
            fire_idx(c0 + 5, qn[2])
        wait_gather(0)
        compute_chunk(0)
        fire_scatter(0, qs[0])
        wait_gather(1)
        compute_chunk(1)
        fire_scatter(1, qs[1])
        wait_gather(2)
        compute_chunk(2)
        fire_scatter(2, qs[2])
        wait_idx(qn[0])
        drain_scatter(0)
        fire_gather(0, qn[0])

    def pair_body(t, carry):
        one_super(2 * t, (0, 1, 2), (3, 4, 5))
        one_super(2 * t + 1, (3, 4, 5), (0, 1, 2))
        return carry

    lax.fori_loop(0, NSUPER // 2, pair_body, 0, unroll=False)
    if NSUPER % 2:
        one_super(NSUPER - 1, (0, 1, 2), (3, 4, 5))
        tail_qs = (3, 4, 5)
    else:
        tail_qs = (0, 1, 2)
    for t in range(NTAIL):
        b = t
        if t + 1 < NTAIL:
            wait_idx(tail_qs[t + 1])
            drain_scatter(t + 1)
            fire_gather(t + 1, tail_qs[t + 1])
        wait_gather(b)
        compute_chunk(b)
        fire_scatter(b, tail_qs[t])
    for b in range(NBUF):
        drain_scatter(b)

    plsc.subcore_barrier()

    @pl.when(sid < NS - 1)
    def _():
        pltpu.sync_copy(accum.at[pl.ds(row0, ROWS_A)],
                        part_hbm.at[cid, pl.ds(row0, ROWS_A)])

    @pl.when(sid == NS - 1)
    def _():
        pltpu.sync_copy(accum.at[pl.ds(15 * ROWS_A, ROWS_LAST)],
                        part_hbm.at[cid, pl.ds(15 * ROWS_A, ROWS_LAST)])

    @pl.when(sid == 0)
    def _():
        pltpu.sync_copy(denom, dpart_hbm.at[cid])


@jax.jit
def _sc_aggregate(h, a, b, src, dst, znd, zn):
    mesh = plsc.VectorSubcoreMesh(core_axis_name="c", subcore_axis_name="s")
    fn = pl.kernel(
        _sc_body,
        out_type=[jax.ShapeDtypeStruct((NC, N, D), jnp.float32),
                  jax.ShapeDtypeStruct((NC, N), jnp.float32)],
        mesh=mesh,
        compiler_params=pltpu.CompilerParams(needs_layout_passes=False),
        scratch_types=[
            [pltpu.VMEM((CHUNK,), jnp.int32) for _ in range(NQ)],
            [pltpu.VMEM((CHUNK,), jnp.int32) for _ in range(NQ)],
            [pltpu.VMEM((CHUNK, D), jnp.float32) for _ in range(NBUF)],
            [pltpu.VMEM((CHUNK,), jnp.float32) for _ in range(NBUF)],
            [pltpu.VMEM((CHUNK,), jnp.float32) for _ in range(NBUF)],
            [pltpu.VMEM((CHUNK,), jnp.float32) for _ in range(NBUF)],
            pltpu.VMEM_SHARED((N, D), jnp.float32),
            pltpu.VMEM_SHARED((N,), jnp.float32),
            pltpu.SemaphoreType.DMA,
            [pltpu.SemaphoreType.DMA for _ in range(NQ)],
            [pltpu.SemaphoreType.DMA for _ in range(NBUF)],
            [pltpu.SemaphoreType.DMA for _ in range(NBUF)],
        ],
    )
    return fn(h, a, b, src, dst, znd, zn)


def _proj_body(x_ref, wn_ref, ws_ref, wab_ref, bias_ref,
               h_ref, s_ref, ab_ref):
    xb = x_ref[...]
    h_ref[...] = jnp.dot(xb, wn_ref[...], preferred_element_type=jnp.float32)
    s_ref[...] = (jnp.dot(xb, ws_ref[...], preferred_element_type=jnp.float32)
                  + bias_ref[...])
    xl = jnp.where(xb >= 0, xb, 0.2 * xb)
    ab_ref[...] = jnp.dot(xl, wab_ref[...], preferred_element_type=jnp.float32)


@jax.jit
def _proj(x, wn_t, ws_t, wab, bias2d):
    return pl.pallas_call(
        _proj_body,
        grid=(GRID,),
        in_specs=[
            pl.BlockSpec((BLK, D), lambda i: (i, 0)),
            pl.BlockSpec((D, D), lambda i: (0, 0)),
            pl.BlockSpec((D, D), lambda i: (0, 0)),
            pl.BlockSpec((D, 2), lambda i: (0, 0)),
            pl.BlockSpec((1, D), lambda i: (0, 0)),
        ],
        out_specs=[
            pl.BlockSpec((BLK, D), lambda i: (i, 0)),
            pl.BlockSpec((BLK, D), lambda i: (i, 0)),
            pl.BlockSpec((BLK, 2), lambda i: (i, 0)),
        ],
        out_shape=[
            jax.ShapeDtypeStruct((N, D), jnp.float32),
            jax.ShapeDtypeStruct((N, D), jnp.float32),
            jax.ShapeDtypeStruct((N, 2), jnp.float32),
        ],
    )(x, wn_t, ws_t, wab, bias2d)


def _onehot(bat):
    io = lax.broadcasted_iota(jnp.int32, (bat.shape[0], G), 1)
    return (bat == io).astype(jnp.float32)


def _combine_stats_body(p_ref, dp_ref, s_ref, batch_ref,
                        y_ref, s1_ref, s2_ref, cnt_ref):
    i = pl.program_id(0)
    pp = p_ref[...]
    dd = dp_ref[...]
    den = jnp.clip(dd[0] + dd[1], 1e-6, None)
    y = (pp[0] + pp[1]) / den + s_ref[...]
    y_ref[...] = y
    oh = _onehot(batch_ref[...])
    dn = (((0,), (0,)), ((), ()))
    s1 = lax.dot_general(oh, y, dn, preferred_element_type=jnp.float32)
    s2 = lax.dot_general(oh, y * y, dn, preferred_element_type=jnp.float32)
    c = lax.dot_general(oh, jnp.ones_like(y), dn,
                        preferred_element_type=jnp.float32)

    @pl.when(i == 0)
    def _():
        s1_ref[...] = jnp.zeros_like(s1_ref)
        s2_ref[...] = jnp.zeros_like(s2_ref)
        cnt_ref[...] = jnp.zeros_like(cnt_ref)

    s1_ref[...] += s1
    s2_ref[...] += s2
    cnt_ref[...] += c


@jax.jit
def _combine_stats(part, dpart3, s, batch2d):
    return pl.pallas_call(
        _combine_stats_body,
        grid=(GRID,),
        in_specs=[
            pl.BlockSpec((NC, BLK, D), lambda i: (0, i, 0)),
            pl.BlockSpec((NC, BLK, 1), lambda i: (0, i, 0)),
            pl.BlockSpec((BLK, D), lambda i: (i, 0)),
            pl.BlockSpec((BLK, 1), lambda i: (i, 0)),
        ],
        out_specs=[
            pl.BlockSpec((BLK, D), lambda i: (i, 0)),
            pl.BlockSpec((G, D), lambda i: (0, 0)),
            pl.BlockSpec((G, D), lambda i: (0, 0)),
            pl.BlockSpec((G, D), lambda i: (0, 0)),
        ],
        out_shape=[
            jax.ShapeDtypeStruct((N, D), jnp.float32),
            jax.ShapeDtypeStruct((G, D), jnp.float32),
            jax.ShapeDtypeStruct((G, D), jnp.float32),
            jax.ShapeDtypeStruct((G, D), jnp.float32),
        ],
    )(part, dpart3, s, batch2d)


def _norm_kan_proj_body(y_ref, batch_ref, s1_ref, s2_ref, cnt_ref,
                        nw_ref, nb_ref, ms_ref, kw_ref, kb_ref,
                        wn_ref, ws_ref, wab_ref, b2_ref,
                        h_ref, so_ref, ab_ref):
    y = y_ref[...]
    oh = _onehot(batch_ref[...])
    cnt = jnp.clip(cnt_ref[...], 1.0, None)
    m = s1_ref[...] / cnt
    ex2 = s2_ref[...] / cnt
    ms = ms_ref[...]
    var = ex2 - m * m * ms * (2.0 - ms)
    std = jnp.sqrt(var + 1e-5)
    mofs = jnp.dot(oh, m * ms, preferred_element_type=jnp.float32)
    sofs = jnp.dot(oh, std, preferred_element_type=jnp.float32)
    t = (y - mofs) / sofs * nw_ref[...] + nb_ref[...]
    kw = kw_ref[...]
    kwe = jnp.exp(kw - jnp.max(kw, axis=0, keepdims=True))
    kwn = kwe / jnp.sum(kwe, axis=0, keepdims=True)
    sig = 1.0 / (1.0 + jnp.exp(-t))
    xn = (kwn[0:1] * t * sig + kwn[1:2] * jnp.tanh(t) + kwn[2:3] * jnp.sin(t)
          + kwn[3:4] * jnp.exp(-0.5 * t * t) + kb_ref[...])
    h_ref[...] = jnp.dot(xn, wn_ref[...], preferred_element_type=jnp.float32)
    so_ref[...] = (jnp.dot(xn, ws_ref[...], preferred_element_type=jnp.float32)
                   + b2_ref[...])
    xl = jnp.where(xn >= 0, xn, 0.2 * xn)
    ab_ref[...] = jnp.dot(xl, wab_ref[...], preferred_element_type=jnp.float32)


@jax.jit
def _norm_kan_proj(y, batch2d, s1, s2, cnt, nw, nb, ms, kw, kb,
                   wn_t, ws_t, wab, bias2d):
    return pl.pallas_call(
        _norm_kan_proj_body,
        grid=(GRID,),
        in_specs=[
            pl.BlockSpec((BLK, D), lambda i: (i, 0)),
            pl.BlockSpec((BLK, 1), lambda i: (i, 0)),
            pl.BlockSpec((G, D), lambda i: (0, 0)),
            pl.BlockSpec((G, D), lambda i: (0, 0)),
            pl.BlockSpec((G, D), lambda i: (0, 0)),
            pl.BlockSpec((1, D), lambda i: (0, 0)),
            pl.BlockSpec((1, D), lambda i: (0, 0)),
            pl.BlockSpec((1, D), lambda i: (0, 0)),
            pl.BlockSpec((4, D), lambda i: (0, 0)),
            pl.BlockSpec((1, D), lambda i: (0, 0)),
            pl.BlockSpec((D, D), lambda i: (0, 0)),
            pl.BlockSpec((D, D), lambda i: (0, 0)),
            pl.BlockSpec((D, 2), lambda i: (0, 0)),
            pl.BlockSpec((1, D), lambda i: (0, 0)),
        ],
        out_specs=[
            pl.BlockSpec((BLK, D), lambda i: (i, 0)),
            pl.BlockSpec((BLK, D), lambda i: (i, 0)),
            pl.BlockSpec((BLK, 2), lambda i: (i, 0)),
        ],
        out_shape=[
            jax.ShapeDtypeStruct((N, D), jnp.float32),
            jax.ShapeDtypeStruct((N, D), jnp.float32),
            jax.ShapeDtypeStruct((N, 2), jnp.float32),
        ],
    )(y, batch2d, s1, s2, cnt, nw, nb, ms, kw, kb, wn_t, ws_t, wab, bias2d)


def _final_body(p_ref, dp_ref, s_ref, y_ref):
    pp = p_ref[...]
    dd = dp_ref[...]
    den = jnp.clip(dd[0] + dd[1], 1e-6, None)
    y_ref[...] = (pp[0] + pp[1]) / den + s_ref[...]


@jax.jit
def _final_combine(part, dpart3, s):
    return pl.pallas_call(
        _final_body,
        grid=(GRID,),
        in_specs=[
            pl.BlockSpec((NC, BLK, D), lambda i: (0, i, 0)),
            pl.BlockSpec((NC, BLK, 1), lambda i: (0, i, 0)),
            pl.BlockSpec((BLK, D), lambda i: (i, 0)),
        ],
        out_specs=pl.BlockSpec((BLK, D), lambda i: (i, 0)),
        out_shape=jax.ShapeDtypeStruct((N, D), jnp.float32),
    )(part, dpart3, s)


def kernel(x, params, edge_index, batch):
    src = edge_index[0]
    dst = edge_index[1]
    batch2d = batch.reshape(N, 1)
    znd = jnp.zeros((N, D), jnp.float32)
    zn = jnp.zeros((N,), jnp.float32)

    def conv_inputs(p):
        wn_t = p['W_neigh'].T
        ws_t = p['W_self'].T
        wab = jnp.stack([p['W_att'][D:], p['W_att'][:D]], axis=1)
        bias2d = p['bias'].reshape(1, D)
        return wn_t, ws_t, wab, bias2d

    convs = params['convs']
    norms = params['norms']
    kans = params['kans']

    wn_t, ws_t, wab, bias2d = conv_inputs(convs[0])
    h, s, ab = _proj(x, wn_t, ws_t, wab, bias2d)

    for l in range(len(convs) - 1):
        b_att = ab[:, 0]
        a_att = ab[:, 1]
        part, dpart = _sc_aggregate(h, a_att, b_att, src, dst, znd, zn)
        dpart3 = dpart.reshape(NC, N, 1)
        y, s1g, s2g, cntg = _combine_stats(part, dpart3, s, batch2d)
        np_ = norms[l]
        kp = kans[l]
        wn_t, ws_t, wab, bias2d = conv_inputs(convs[l + 1])
        h, s, ab = _norm_kan_proj(
            y, batch2d, s1g, s2g, cntg,
            np_['weight'].reshape(1, D), np_['bias'].reshape(1, D),
            np_['mean_scale'].reshape(1, D),
            kp['weights'].T, kp['bias'].reshape(1, D),
            wn_t, ws_t, wab, bias2d)

    b_att = ab[:, 0]
    a_att = ab[:, 1]
    part, dpart = _sc_aggregate(h, a_att, b_att, src, dst, znd, zn)
    return _final_combine(part, dpart.reshape(NC, N, 1), s)


# speedup vs baseline: 1.0240x; 1.0240x over previous
import functools

import jax
import jax.numpy as jnp
from jax import lax
from jax.experimental import pallas as pl
from jax.experimental.pallas import tpu as pltpu
from jax.experimental.pallas import tpu_sc as plsc

N = 10000
E = 320000
D = 128
G = 16

NC = 2
NS = 16
NW = NC * NS
EPW = E // NW
CHUNK = 80
NCHUNK = EPW // CHUNK
NGROUP = CHUNK // 16
NBUF = 3
NQ = 2 * NBUF
NSUPER = NCHUNK // NBUF
NTAIL = NCHUNK - NSUPER * NBUF

ROWS_A = 624
ROWS_LAST = N - 15 * ROWS_A

BLK = 2000
GRID = N // BLK


def _sc_body(h_hbm, ab_hbm, src_hbm, dst_hbm, znd_hbm, zn_hbm,
             part_hbm, dpart_hbm,
             ab_v, src_v, dst_v, rows_v, alpha_v,
             accum, denom, stage_sem, isem, gsem, ssem):
    cid = lax.axis_index("c")
    sid = lax.axis_index("s")
    wid = cid * NS + sid
    base_e = wid * EPW

    row0 = pl.multiple_of(sid * ROWS_A, 8)

    @pl.when(sid < NS - 1)
    def _():
        pltpu.async_copy(znd_hbm.at[pl.ds(row0, ROWS_A)],
                         accum.at[pl.ds(row0, ROWS_A)], stage_sem).wait()

    @pl.when(sid == NS - 1)
    def _():
        pltpu.async_copy(znd_hbm.at[pl.ds(15 * ROWS_A, ROWS_LAST)],
                         accum.at[pl.ds(15 * ROWS_A, ROWS_LAST)],
                         stage_sem).wait()

    @pl.when(sid == 0)
    def _():
        pltpu.async_copy(zn_hbm, denom, stage_sem).wait()

    pltpu.async_copy(ab_hbm, ab_v, stage_sem).wait()
    plsc.subcore_barrier()

    def fire_idx(c, q):
        cb = pl.multiple_of(base_e + c * CHUNK, 8)
        pltpu.async_copy(src_hbm.at[pl.ds(cb, CHUNK)], src_v[q], isem[q])
        pltpu.async_copy(dst_hbm.at[pl.ds(cb, CHUNK)], dst_v[q], isem[q])

    def wait_idx(q):
        pltpu.make_async_copy(src_hbm.at[pl.ds(0, CHUNK)], src_v[q],
                              isem[q]).wait()
        pltpu.make_async_copy(dst_hbm.at[pl.ds(0, CHUNK)], dst_v[q],
                              isem[q]).wait()

    def fire_gather(b, q):
        pltpu.async_copy(h_hbm.at[src_v[q]], rows_v[b], gsem[b])

    def wait_gather(b):
        pltpu.make_async_copy(znd_hbm.at[pl.ds(0, CHUNK)], rows_v[b],
                              gsem[b]).wait()

    def drain_scatter(b):
        pltpu.make_async_copy(znd_hbm.at[pl.ds(0, CHUNK)], rows_v[b],
                              ssem[b]).wait()
        pltpu.make_async_copy(zn_hbm.at[pl.ds(0, CHUNK)], alpha_v[b],
                              ssem[b]).wait()

    def compute_chunk(b, q):
        def group_body(g, carry2):
            e0 = g * 16
            s16 = src_v[q][pl.ds(e0, 16)]
            d16 = dst_v[q][pl.ds(e0, 16)]
            pd = plsc.load_gather(ab_v, [d16])
            ps = plsc.load_gather(ab_v, [s16])
            av = plsc.bitcast(lax.shift_left(pd, 16), jnp.float32)
            bv = plsc.bitcast(
                jnp.bitwise_and(ps, jnp.int32(-65536)), jnp.float32)
            alpha = 1.0 / (1.0 + jnp.exp(-(av + bv)))
            alpha_v[b][pl.ds(e0, 16)] = alpha
            for k in range(16):
                e = e0 + k
                asp = plsc.load_gather(
                    alpha_v[b],
                    [jnp.broadcast_to(e, (16,)).astype(jnp.int32)])
                for r in range(D // 16):
                    sl = pl.ds(r * 16, 16)
                    rows_v[b][e, sl] = rows_v[b][e, sl] * asp
            return carry2

        lax.fori_loop(0, NGROUP, group_body, 0, unroll=False)

    def fire_scatter(b, q):
        pltpu.async_copy(rows_v[b], accum.at[dst_v[q]], ssem[b], add=True)
        pltpu.async_copy(alpha_v[b], denom.at[dst_v[q]], ssem[b], add=True)

    for c in range(NBUF):
        fire_idx(c, c)
    for b in range(NBUF):
        pltpu.async_copy(znd_hbm.at[pl.ds(0, CHUNK)], rows_v[b], ssem[b])
        pltpu.async_copy(zn_hbm.at[pl.ds(0, CHUNK)], alpha_v[b], ssem[b])
    wait_idx(0)
    drain_scatter(0)
    fire_gather(0, 0)

    def one_super(j, qs, qn):
        c0 = j * NBUF
        wait_idx(qs[1])
        drain_scatter(1)
        fire_gather(1, qs[1])
        wait_idx(qs[2])
        drain_scatter(2)
        fire_gather(2, qs[2])
        fire_idx(c0 + 3, qn[0])
        fire_idx(c0 + 4, qn[1])

        @pl.when(c0 + 5 < NCHUNK)
        def _():
            fire_idx(c0 + 5, qn[2])
        wait_gather(0)
        compute_chunk(0, qs[0])
        fire_scatter(0, qs[0])
        wait_gather(1)
        compute_chunk(1, qs[1])
        fire_scatter(1, qs[1])
        wait_gather(2)
        compute_chunk(2, qs[2])
        fire_scatter(2, qs[2])
        wait_idx(qn[0])
        drain_scatter(0)
        fire_gather(0, qn[0])

    def pair_body(t, carry):
        one_super(2 * t, (0, 1, 2), (3, 4, 5))
        one_super(2 * t + 1, (3, 4, 5), (0, 1, 2))
        return carry

    lax.fori_loop(0, NSUPER // 2, pair_body, 0, unroll=False)
    if NSUPER % 2:
        one_super(NSUPER - 1, (0, 1, 2), (3, 4, 5))
        tail_qs = (3, 4, 5)
    else:
        tail_qs = (0, 1, 2)
    for t in range(NTAIL):
        b = t
        if t + 1 < NTAIL:
            wait_idx(tail_qs[t + 1])
            drain_scatter(t + 1)
            fire_gather(t + 1, tail_qs[t + 1])
        wait_gather(b)
        compute_chunk(b, tail_qs[t])
        fire_scatter(b, tail_qs[t])
    for b in range(NBUF):
        drain_scatter(b)

    plsc.subcore_barrier()

    @pl.when(sid < NS - 1)
    def _():
        pltpu.sync_copy(accum.at[pl.ds(row0, ROWS_A)],
                        part_hbm.at[cid, pl.ds(row0, ROWS_A)])

    @pl.when(sid == NS - 1)
    def _():
        pltpu.sync_copy(accum.at[pl.ds(15 * ROWS_A, ROWS_LAST)],
                        part_hbm.at[cid, pl.ds(15 * ROWS_A, ROWS_LAST)])

    @pl.when(sid == 0)
    def _():
        pltpu.sync_copy(denom, dpart_hbm.at[cid])


@jax.jit
def _sc_aggregate(h, ab, src, dst, znd, zn):
    mesh = plsc.VectorSubcoreMesh(core_axis_name="c", subcore_axis_name="s")
    fn = pl.kernel(
        _sc_body,
        out_type=[jax.ShapeDtypeStruct((NC, N, D), jnp.float32),
                  jax.ShapeDtypeStruct((NC, N), jnp.float32)],
        mesh=mesh,
        compiler_params=pltpu.CompilerParams(needs_layout_passes=False),
        scratch_types=[
            pltpu.VMEM((N,), jnp.int32),
            [pltpu.VMEM((CHUNK,), jnp.int32) for _ in range(NQ)],
            [pltpu.VMEM((CHUNK,), jnp.int32) for _ in range(NQ)],
            [pltpu.VMEM((CHUNK, D), jnp.float32) for _ in range(NBUF)],
            [pltpu.VMEM((CHUNK,), jnp.float32) for _ in range(NBUF)],
            pltpu.VMEM_SHARED((N, D), jnp.float32),
            pltpu.VMEM_SHARED((N,), jnp.float32),
            pltpu.SemaphoreType.DMA,
            [pltpu.SemaphoreType.DMA for _ in range(NQ)],
            [pltpu.SemaphoreType.DMA for _ in range(NBUF)],
            [pltpu.SemaphoreType.DMA for _ in range(NBUF)],
        ],
    )
    return fn(h, ab, src, dst, znd, zn)


def _proj_body(x_ref, wn_ref, ws_ref, wab_ref, bias_ref,
               h_ref, s_ref, ab_ref):
    xb = x_ref[...]
    h_ref[...] = jnp.dot(xb, wn_ref[...], preferred_element_type=jnp.float32)
    s_ref[...] = (jnp.dot(xb, ws_ref[...], preferred_element_type=jnp.float32)
                  + bias_ref[...])
    xl = jnp.where(xb >= 0, xb, 0.2 * xb)
    ab_ref[...] = jnp.dot(xl, wab_ref[...], preferred_element_type=jnp.float32)


@jax.jit
def _proj(x, wn_t, ws_t, wab, bias2d):
    return pl.pallas_call(
        _proj_body,
        grid=(GRID,),
        in_specs=[
            pl.BlockSpec((BLK, D), lambda i: (i, 0)),
            pl.BlockSpec((D, D), lambda i: (0, 0)),
            pl.BlockSpec((D, D), lambda i: (0, 0)),
            pl.BlockSpec((D, 2), lambda i: (0, 0)),
            pl.BlockSpec((1, D), lambda i: (0, 0)),
        ],
        out_specs=[
            pl.BlockSpec((BLK, D), lambda i: (i, 0)),
            pl.BlockSpec((BLK, D), lambda i: (i, 0)),
            pl.BlockSpec((BLK, 2), lambda i: (i, 0)),
        ],
        out_shape=[
            jax.ShapeDtypeStruct((N, D), jnp.float32),
            jax.ShapeDtypeStruct((N, D), jnp.float32),
            jax.ShapeDtypeStruct((N, 2), jnp.float32),
        ],
    )(x, wn_t, ws_t, wab, bias2d)


def _onehot(bat):
    io = lax.broadcasted_iota(jnp.int32, (bat.shape[0], G), 1)
    return (bat == io).astype(jnp.float32)


def _combine_stats_body(p_ref, dp_ref, s_ref, batch_ref,
                        y_ref, s1_ref, s2_ref, cnt_ref):
    i = pl.program_id(0)
    pp = p_ref[...]
    dd = dp_ref[...]
    den = jnp.clip(dd[0] + dd[1], 1e-6, None)
    y = (pp[0] + pp[1]) / den + s_ref[...]
    y_ref[...] = y
    oh = _onehot(batch_ref[...])
    dn = (((0,), (0,)), ((), ()))
    s1 = lax.dot_general(oh, y, dn, preferred_element_type=jnp.float32)
    s2 = lax.dot_general(oh, y * y, dn, preferred_element_type=jnp.float32)
    c = lax.dot_general(oh, jnp.ones_like(y), dn,
                        preferred_element_type=jnp.float32)

    @pl.when(i == 0)
    def _():
        s1_ref[...] = jnp.zeros_like(s1_ref)
        s2_ref[...] = jnp.zeros_like(s2_ref)
        cnt_ref[...] = jnp.zeros_like(cnt_ref)

    s1_ref[...] += s1
    s2_ref[...] += s2
    cnt_ref[...] += c


@jax.jit
def _combine_stats(part, dpart3, s, batch2d):
    return pl.pallas_call(
        _combine_stats_body,
        grid=(GRID,),
        in_specs=[
            pl.BlockSpec((NC, BLK, D), lambda i: (0, i, 0)),
            pl.BlockSpec((NC, BLK, 1), lambda i: (0, i, 0)),
            pl.BlockSpec((BLK, D), lambda i: (i, 0)),
            pl.BlockSpec((BLK, 1), lambda i: (i, 0)),
        ],
        out_specs=[
            pl.BlockSpec((BLK, D), lambda i: (i, 0)),
            pl.BlockSpec((G, D), lambda i: (0, 0)),
            pl.BlockSpec((G, D), lambda i: (0, 0)),
            pl.BlockSpec((G, D), lambda i: (0, 0)),
        ],
        out_shape=[
            jax.ShapeDtypeStruct((N, D), jnp.float32),
            jax.ShapeDtypeStruct((G, D), jnp.float32),
            jax.ShapeDtypeStruct((G, D), jnp.float32),
            jax.ShapeDtypeStruct((G, D), jnp.float32),
        ],
    )(part, dpart3, s, batch2d)


def _norm_kan_proj_body(y_ref, batch_ref, s1_ref, s2_ref, cnt_ref,
                        nw_ref, nb_ref, ms_ref, kw_ref, kb_ref,
                        wn_ref, ws_ref, wab_ref, b2_ref,
                        h_ref, so_ref, ab_ref):
    y = y_ref[...]
    oh = _onehot(batch_ref[...])
    cnt = jnp.clip(cnt_ref[...], 1.0, None)
    m = s1_ref[...] / cnt
    ex2 = s2_ref[...] / cnt
    ms = ms_ref[...]
    var = ex2 - m * m * ms * (2.0 - ms)
    std = jnp.sqrt(var + 1e-5)
    mofs = jnp.dot(oh, m * ms, preferred_element_type=jnp.float32)
    sofs = jnp.dot(oh, std, preferred_element_type=jnp.float32)
    t = (y - mofs) / sofs * nw_ref[...] + nb_ref[...]
    kw = kw_ref[...]
    kwe = jnp.exp(kw - jnp.max(kw, axis=0, keepdims=True))
    kwn = kwe / jnp.sum(kwe, axis=0, keepdims=True)
    sig = 1.0 / (1.0 + jnp.exp(-t))
    xn = (kwn[0:1] * t * sig + kwn[1:2] * jnp.tanh(t) + kwn[2:3] * jnp.sin(t)
          + kwn[3:4] * jnp.exp(-0.5 * t * t) + kb_ref[...])
    h_ref[...] = jnp.dot(xn, wn_ref[...], preferred_element_type=jnp.float32)
    so_ref[...] = (jnp.dot(xn, ws_ref[...], preferred_element_type=jnp.float32)
                   + b2_ref[...])
    xl = jnp.where(xn >= 0, xn, 0.2 * xn)
    ab_ref[...] = jnp.dot(xl, wab_ref[...], preferred_element_type=jnp.float32)


@jax.jit
def _norm_kan_proj(y, batch2d, s1, s2, cnt, nw, nb, ms, kw, kb,
                   wn_t, ws_t, wab, bias2d):
    return pl.pallas_call(
        _norm_kan_proj_body,
        grid=(GRID,),
        in_specs=[
            pl.BlockSpec((BLK, D), lambda i: (i, 0)),
            pl.BlockSpec((BLK, 1), lambda i: (i, 0)),
            pl.BlockSpec((G, D), lambda i: (0, 0)),
            pl.BlockSpec((G, D), lambda i: (0, 0)),
            pl.BlockSpec((G, D), lambda i: (0, 0)),
            pl.BlockSpec((1, D), lambda i: (0, 0)),
            pl.BlockSpec((1, D), lambda i: (0, 0)),
            pl.BlockSpec((1, D), lambda i: (0, 0)),
            pl.BlockSpec((4, D), lambda i: (0, 0)),
            pl.BlockSpec((1, D), lambda i: (0, 0)),
            pl.BlockSpec((D, D), lambda i: (0, 0)),
            pl.BlockSpec((D, D), lambda i: (0, 0)),
            pl.BlockSpec((D, 2), lambda i: (0, 0)),
            pl.BlockSpec((1, D), lambda i: (0, 0)),
        ],
        out_specs=[
            pl.BlockSpec((BLK, D), lambda i: (i, 0)),
            pl.BlockSpec((BLK, D), lambda i: (i, 0)),
            pl.BlockSpec((BLK, 2), lambda i: (i, 0)),
        ],
        out_shape=[
            jax.ShapeDtypeStruct((N, D), jnp.float32),
            jax.ShapeDtypeStruct((N, D), jnp.float32),
            jax.ShapeDtypeStruct((N, 2), jnp.float32),
        ],
    )(y, batch2d, s1, s2, cnt, nw, nb, ms, kw, kb, wn_t, ws_t, wab, bias2d)


def _final_body(p_ref, dp_ref, s_ref, y_ref):
    pp = p_ref[...]
    dd = dp_ref[...]
    den = jnp.clip(dd[0] + dd[1], 1e-6, None)
    y_ref[...] = (pp[0] + pp[1]) / den + s_ref[...]


@jax.jit
def _final_combine(part, dpart3, s):
    return pl.pallas_call(
        _final_body,
        grid=(GRID,),
        in_specs=[
            pl.BlockSpec((NC, BLK, D), lambda i: (0, i, 0)),
            pl.BlockSpec((NC, BLK, 1), lambda i: (0, i, 0)),
            pl.BlockSpec((BLK, D), lambda i: (i, 0)),
        ],
        out_specs=pl.BlockSpec((BLK, D), lambda i: (i, 0)),
        out_shape=jax.ShapeDtypeStruct((N, D), jnp.float32),
    )(part, dpart3, s)


def _pack_ab(a, b):
    a16 = jax.lax.bitcast_convert_type(a.astype(jnp.bfloat16),
                                       jnp.uint16).astype(jnp.uint32)
    b16 = jax.lax.bitcast_convert_type(b.astype(jnp.bfloat16),
                                       jnp.uint16).astype(jnp.uint32)
    return jax.lax.bitcast_convert_type(a16 | (b16 << 16), jnp.int32)


def kernel(x, params, edge_index, batch):
    src = edge_index[0]
    dst = edge_index[1]
    batch2d = batch.reshape(N, 1)
    znd = jnp.zeros((N, D), jnp.float32)
    zn = jnp.zeros((N,), jnp.float32)

    def conv_inputs(p):
        wn_t = p['W_neigh'].T
        ws_t = p['W_self'].T
        wab = jnp.stack([p['W_att'][D:], p['W_att'][:D]], axis=1)
        bias2d = p['bias'].reshape(1, D)
        return wn_t, ws_t, wab, bias2d

    convs = params['convs']
    norms = params['norms']
    kans = params['kans']

    wn_t, ws_t, wab, bias2d = conv_inputs(convs[0])
    h, s, ab = _proj(x, wn_t, ws_t, wab, bias2d)

    for l in range(len(convs) - 1):
        packed = _pack_ab(ab[:, 1], ab[:, 0])
        part, dpart = _sc_aggregate(h, packed, src, dst, znd, zn)
        dpart3 = dpart.reshape(NC, N, 1)
        y, s1g, s2g, cntg = _combine_stats(part, dpart3, s, batch2d)
        np_ = norms[l]
        kp = kans[l]
        wn_t, ws_t, wab, bias2d = conv_inputs(convs[l + 1])
        h, s, ab = _norm_kan_proj(
            y, batch2d, s1g, s2g, cntg,
            np_['weight'].reshape(1, D), np_['bias'].reshape(1, D),
            np_['mean_scale'].reshape(1, D),
            kp['weights'].T, kp['bias'].reshape(1, D),
            wn_t, ws_t, wab, bias2d)

    packed = _pack_ab(ab[:, 1], ab[:, 0])
    part, dpart = _sc_aggregate(h, packed, src, dst, znd, zn)
    return _final_combine(part, dpart.reshape(NC, N, 1), s)


# speedup vs baseline: 1.0686x; 1.0435x over previous
import functools

import jax
import jax.numpy as jnp
from jax import lax
from jax.experimental import pallas as pl
from jax.experimental.pallas import tpu as pltpu
from jax.experimental.pallas import tpu_sc as plsc

N = 10000
E = 320000
D = 128
G = 16

NC = 2
NS = 16
NW = NC * NS
EPW = E // NW
CHUNK = 80
NCHUNK = EPW // CHUNK
NGROUP = CHUNK // 16
NBUF = 2
NSUPER = NCHUNK // NBUF
NTAIL = NCHUNK - NSUPER * NBUF

ROWS_A = 624
ROWS_LAST = N - 15 * ROWS_A

BLK = 2000
GRID = N // BLK


def _sc_body(h_hbm, a_hbm, b_hbm, src_hbm, dst_hbm, znd_hbm, zn_hbm,
             part_hbm, dpart_hbm,
             a_v, b_v, src_v, dst_v, rows_v, alpha_v,
             accum, denom, stage_sem, isem, gsem, ssem):
    cid = lax.axis_index("c")
    sid = lax.axis_index("s")
    wid = cid * NS + sid
    base_e = wid * EPW

    row0 = pl.multiple_of(sid * ROWS_A, 8)

    @pl.when(sid < NS - 1)
    def _():
        pltpu.async_copy(znd_hbm.at[pl.ds(row0, ROWS_A)],
                         accum.at[pl.ds(row0, ROWS_A)], stage_sem).wait()

    @pl.when(sid == NS - 1)
    def _():
        pltpu.async_copy(znd_hbm.at[pl.ds(15 * ROWS_A, ROWS_LAST)],
                         accum.at[pl.ds(15 * ROWS_A, ROWS_LAST)],
                         stage_sem).wait()

    @pl.when(sid == 0)
    def _():
        pltpu.async_copy(zn_hbm, denom, stage_sem).wait()

    cp_a = pltpu.async_copy(a_hbm, a_v, stage_sem)
    cp_b = pltpu.async_copy(b_hbm, b_v, stage_sem)
    cp_a.wait()
    cp_b.wait()
    plsc.subcore_barrier()

    def fire_idx(c, q):
        cb = pl.multiple_of(base_e + c * CHUNK, 8)
        return (pltpu.async_copy(src_hbm.at[pl.ds(cb, CHUNK)], src_v[q],
                                 isem[q]),
                pltpu.async_copy(dst_hbm.at[pl.ds(cb, CHUNK)], dst_v[q],
                                 isem[q]))

    def fire_gather(b, q):
        pltpu.async_copy(h_hbm.at[src_v[q]], rows_v[b], gsem[b])

    def wait_gather(b):
        pltpu.make_async_copy(znd_hbm.at[pl.ds(0, CHUNK)], rows_v[b],
                              gsem[b]).wait()

    def drain_scatter(b):
        pltpu.make_async_copy(znd_hbm.at[pl.ds(0, CHUNK)], rows_v[b],
                              ssem[b]).wait()
        pltpu.make_async_copy(zn_hbm.at[pl.ds(0, CHUNK)], alpha_v[b],
                              ssem[b]).wait()

    def compute_chunk(b, q):
        def group_body(g, carry2):
            e0 = g * 16
            s16 = src_v[q][pl.ds(e0, 16)]
            d16 = dst_v[q][pl.ds(e0, 16)]
            av = plsc.load_gather(a_v, [d16])
            bv = plsc.load_gather(b_v, [s16])
            alpha = 1.0 / (1.0 + jnp.exp(-(av + bv)))
            alpha_v[b][pl.ds(e0, 16)] = alpha
            for k in range(16):
                e = e0 + k
                asp = plsc.load_gather(
                    alpha_v[b],
                    [jnp.broadcast_to(e, (16,)).astype(jnp.int32)])
                for r in range(D // 16):
                    sl = pl.ds(r * 16, 16)
                    rows_v[b][e, sl] = rows_v[b][e, sl] * asp
            return carry2

        lax.fori_loop(0, NGROUP, group_body, 0, unroll=False)

    def fire_scatter(b, q):
        pltpu.async_copy(rows_v[b], accum.at[dst_v[q]], ssem[b], add=True)
        pltpu.async_copy(alpha_v[b], denom.at[dst_v[q]], ssem[b], add=True)

    fire_idx(0, 0)
    fire_idx(1, 1)
    for b in range(NBUF):
        pltpu.async_copy(znd_hbm.at[pl.ds(0, CHUNK)], rows_v[b], ssem[b])
        pltpu.async_copy(zn_hbm.at[pl.ds(0, CHUNK)], alpha_v[b], ssem[b])

    def wait_idx(q):
        pltpu.make_async_copy(src_hbm.at[pl.ds(0, CHUNK)], src_v[q],
                              isem[q]).wait()
        pltpu.make_async_copy(dst_hbm.at[pl.ds(0, CHUNK)], dst_v[q],
                              isem[q]).wait()

    def one_super(j, q0, q1, qn0, qn1):
        c0 = j * NBUF
        wait_idx(q0)
        drain_scatter(0)
        fire_gather(0, q0)
        wait_idx(q1)
        drain_scatter(1)
        fire_gather(1, q1)
        fire_idx(c0 + 2, qn0)

        @pl.when(c0 + 3 < NCHUNK)
        def _():
            fire_idx(c0 + 3, qn1)
        wait_gather(0)
        compute_chunk(0, q0)
        fire_scatter(0, q0)
        wait_gather(1)
        compute_chunk(1, q1)
        fire_scatter(1, q1)

    def pair_body(t, carry):
        one_super(2 * t, 0, 1, 2, 3)
        one_super(2 * t + 1, 2, 3, 0, 1)
        return carry

    lax.fori_loop(0, NSUPER // 2, pair_body, 0, unroll=False)
    wait_idx(0)
    drain_scatter(0)
    fire_gather(0, 0)
    drain_scatter(1)
    wait_gather(0)
    compute_chunk(0, 0)
    fire_scatter(0, 0)
    drain_scatter(0)

    plsc.subcore_barrier()

    @pl.when(sid < NS - 1)
    def _():
        pltpu.sync_copy(accum.at[pl.ds(row0, ROWS_A)],
                        part_hbm.at[cid, pl.ds(row0, ROWS_A)])

    @pl.when(sid == NS - 1)
    def _():
        pltpu.sync_copy(accum.at[pl.ds(15 * ROWS_A, ROWS_LAST)],
                        part_hbm.at[cid, pl.ds(15 * ROWS_A, ROWS_LAST)])

    @pl.when(sid == 0)
    def _():
        pltpu.sync_copy(denom, dpart_hbm.at[cid])


@jax.jit
def _sc_aggregate(h, a, b, src, dst, znd, zn):
    mesh = plsc.VectorSubcoreMesh(core_axis_name="c", subcore_axis_name="s")
    fn = pl.kernel(
        _sc_body,
        out_type=[jax.ShapeDtypeStruct((NC, N, D), jnp.float32),
                  jax.ShapeDtypeStruct((NC, N), jnp.float32)],
        mesh=mesh,
        compiler_params=pltpu.CompilerParams(needs_layout_passes=False),
        scratch_types=[
            pltpu.VMEM((N,), jnp.float32),
            pltpu.VMEM((N,), jnp.float32),
            [pltpu.VMEM((CHUNK,), jnp.int32) for _ in range(4)],
            [pltpu.VMEM((CHUNK,), jnp.int32) for _ in range(4)],
            [pltpu.VMEM((CHUNK, D), jnp.float32) for _ in range(NBUF)],
            [pltpu.VMEM((CHUNK,), jnp.float32) for _ in range(NBUF)],
            pltpu.VMEM_SHARED((N, D), jnp.float32),
            pltpu.VMEM_SHARED((N,), jnp.float32),
            pltpu.SemaphoreType.DMA,
            [pltpu.SemaphoreType.DMA for _ in range(4)],
            [pltpu.SemaphoreType.DMA for _ in range(NBUF)],
            [pltpu.SemaphoreType.DMA for _ in range(NBUF)],
        ],
    )
    return fn(h, a, b, src, dst, znd, zn)


def _proj_body(x_ref, wn_ref, ws_ref, wab_ref, bias_ref,
               h_ref, s_ref, ab_ref):
    xb = x_ref[...]
    h_ref[...] = jnp.dot(xb, wn_ref[...], preferred_element_type=jnp.float32)
    s_ref[...] = (jnp.dot(xb, ws_ref[...], preferred_element_type=jnp.float32)
                  + bias_ref[...])
    xl = jnp.where(xb >= 0, xb, 0.2 * xb)
    ab_ref[...] = jnp.dot(xl, wab_ref[...], preferred_element_type=jnp.float32)


@jax.jit
def _proj(x, wn_t, ws_t, wab, bias2d):
    return pl.pallas_call(
        _proj_body,
        grid=(GRID,),
        in_specs=[
            pl.BlockSpec((BLK, D), lambda i: (i, 0)),
            pl.BlockSpec((D, D), lambda i: (0, 0)),
            pl.BlockSpec((D, D), lambda i: (0, 0)),
            pl.BlockSpec((D, 2), lambda i: (0, 0)),
            pl.BlockSpec((1, D), lambda i: (0, 0)),
        ],
        out_specs=[
            pl.BlockSpec((BLK, D), lambda i: (i, 0)),
            pl.BlockSpec((BLK, D), lambda i: (i, 0)),
            pl.BlockSpec((BLK, 2), lambda i: (i, 0)),
        ],
        out_shape=[
            jax.ShapeDtypeStruct((N, D), jnp.float32),
            jax.ShapeDtypeStruct((N, D), jnp.float32),
            jax.ShapeDtypeStruct((N, 2), jnp.float32),
        ],
    )(x, wn_t, ws_t, wab, bias2d)


def _onehot(bat):
    io = lax.broadcasted_iota(jnp.int32, (bat.shape[0], G), 1)
    return (bat == io).astype(jnp.float32)


def _combine_stats_body(p_ref, dp_ref, s_ref, batch_ref,
                        y_ref, s1_ref, s2_ref, cnt_ref):
    i = pl.program_id(0)
    pp = p_ref[...]
    dd = dp_ref[...]
    den = jnp.clip(dd[0] + dd[1], 1e-6, None)
    y = (pp[0] + pp[1]) / den + s_ref[...]
    y_ref[...] = y
    oh = _onehot(batch_ref[...])
    dn = (((0,), (0,)), ((), ()))
    s1 = lax.dot_general(oh, y, dn, preferred_element_type=jnp.float32)
    s2 = lax.dot_general(oh, y * y, dn, preferred_element_type=jnp.float32)
    c = lax.dot_general(oh, jnp.ones_like(y), dn,
                        preferred_element_type=jnp.float32)

    @pl.when(i == 0)
    def _():
        s1_ref[...] = jnp.zeros_like(s1_ref)
        s2_ref[...] = jnp.zeros_like(s2_ref)
        cnt_ref[...] = jnp.zeros_like(cnt_ref)

    s1_ref[...] += s1
    s2_ref[...] += s2
    cnt_ref[...] += c


@jax.jit
def _combine_stats(part, dpart3, s, batch2d):
    return pl.pallas_call(
        _combine_stats_body,
        grid=(GRID,),
        in_specs=[
            pl.BlockSpec((NC, BLK, D), lambda i: (0, i, 0)),
            pl.BlockSpec((NC, BLK, 1), lambda i: (0, i, 0)),
            pl.BlockSpec((BLK, D), lambda i: (i, 0)),
            pl.BlockSpec((BLK, 1), lambda i: (i, 0)),
        ],
        out_specs=[
            pl.BlockSpec((BLK, D), lambda i: (i, 0)),
            pl.BlockSpec((G, D), lambda i: (0, 0)),
            pl.BlockSpec((G, D), lambda i: (0, 0)),
            pl.BlockSpec((G, D), lambda i: (0, 0)),
        ],
        out_shape=[
            jax.ShapeDtypeStruct((N, D), jnp.float32),
            jax.ShapeDtypeStruct((G, D), jnp.float32),
            jax.ShapeDtypeStruct((G, D), jnp.float32),
            jax.ShapeDtypeStruct((G, D), jnp.float32),
        ],
    )(part, dpart3, s, batch2d)


def _norm_kan_proj_body(y_ref, batch_ref, s1_ref, s2_ref, cnt_ref,
                        nw_ref, nb_ref, ms_ref, kw_ref, kb_ref,
                        wn_ref, ws_ref, wab_ref, b2_ref,
                        h_ref, so_ref, ab_ref):
    y = y_ref[...]
    oh = _onehot(batch_ref[...])
    cnt = jnp.clip(cnt_ref[...], 1.0, None)
    m = s1_ref[...] / cnt
    ex2 = s2_ref[...] / cnt
    ms = ms_ref[...]
    var = ex2 - m * m * ms * (2.0 - ms)
    std = jnp.sqrt(var + 1e-5)
    mofs = jnp.dot(oh, m * ms, preferred_element_type=jnp.float32)
    sofs = jnp.dot(oh, std, preferred_element_type=jnp.float32)
    t = (y - mofs) / sofs * nw_ref[...] + nb_ref[...]
    kw = kw_ref[...]
    kwe = jnp.exp(kw - jnp.max(kw, axis=0, keepdims=True))
    kwn = kwe / jnp.sum(kwe, axis=0, keepdims=True)
    sig = 1.0 / (1.0 + jnp.exp(-t))
    xn = (kwn[0:1] * t * sig + kwn[1:2] * jnp.tanh(t) + kwn[2:3] * jnp.sin(t)
          + kwn[3:4] * jnp.exp(-0.5 * t * t) + kb_ref[...])
    h_ref[...] = jnp.dot(xn, wn_ref[...], preferred_element_type=jnp.float32)
    so_ref[...] = (jnp.dot(xn, ws_ref[...], preferred_element_type=jnp.float32)
                   + b2_ref[...])
    xl = jnp.where(xn >= 0, xn, 0.2 * xn)
    ab_ref[...] = jnp.dot(xl, wab_ref[...], preferred_element_type=jnp.float32)


@jax.jit
def _norm_kan_proj(y, batch2d, s1, s2, cnt, nw, nb, ms, kw, kb,
                   wn_t, ws_t, wab, bias2d):
    return pl.pallas_call(
        _norm_kan_proj_body,
        grid=(GRID,),
        in_specs=[
            pl.BlockSpec((BLK, D), lambda i: (i, 0)),
            pl.BlockSpec((BLK, 1), lambda i: (i, 0)),
            pl.BlockSpec((G, D), lambda i: (0, 0)),
            pl.BlockSpec((G, D), lambda i: (0, 0)),
            pl.BlockSpec((G, D), lambda i: (0, 0)),
            pl.BlockSpec((1, D), lambda i: (0, 0)),
            pl.BlockSpec((1, D), lambda i: (0, 0)),
            pl.BlockSpec((1, D), lambda i: (0, 0)),
            pl.BlockSpec((4, D), lambda i: (0, 0)),
            pl.BlockSpec((1, D), lambda i: (0, 0)),
            pl.BlockSpec((D, D), lambda i: (0, 0)),
            pl.BlockSpec((D, D), lambda i: (0, 0)),
            pl.BlockSpec((D, 2), lambda i: (0, 0)),
            pl.BlockSpec((1, D), lambda i: (0, 0)),
        ],
        out_specs=[
            pl.BlockSpec((BLK, D), lambda i: (i, 0)),
            pl.BlockSpec((BLK, D), lambda i: (i, 0)),
            pl.BlockSpec((BLK, 2), lambda i: (i, 0)),
        ],
        out_shape=[
            jax.ShapeDtypeStruct((N, D), jnp.float32),
            jax.ShapeDtypeStruct((N, D), jnp.float32),
            jax.ShapeDtypeStruct((N, 2), jnp.float32),
        ],
    )(y, batch2d, s1, s2, cnt, nw, nb, ms, kw, kb, wn_t, ws_t, wab, bias2d)


def _final_body(p_ref, dp_ref, s_ref, y_ref):
    pp = p_ref[...]
    dd = dp_ref[...]
    den = jnp.clip(dd[0] + dd[1], 1e-6, None)
    y_ref[...] = (pp[0] + pp[1]) / den + s_ref[...]


@jax.jit
def _final_combine(part, dpart3, s):
    return pl.pallas_call(
        _final_body,
        grid=(GRID,),
        in_specs=[
            pl.BlockSpec((NC, BLK, D), lambda i: (0, i, 0)),
            pl.BlockSpec((NC, BLK, 1), lambda i: (0, i, 0)),
            pl.BlockSpec((BLK, D), lambda i: (i, 0)),
        ],
        out_specs=pl.BlockSpec((BLK, D), lambda i: (i, 0)),
        out_shape=jax.ShapeDtypeStruct((N, D), jnp.float32),
    )(part, dpart3, s)


def kernel(x, params, edge_index, batch):
    src = edge_index[0]
    dst = edge_index[1]
    batch2d = batch.reshape(N, 1)
    znd = jnp.zeros((N, D), jnp.float32)
    zn = jnp.zeros((N,), jnp.float32)

    def conv_inputs(p):
        wn_t = p['W_neigh'].T
        ws_t = p['W_self'].T
        wab = jnp.stack([p['W_att'][D:], p['W_att'][:D]], axis=1)
        bias2d = p['bias'].reshape(1, D)
        return wn_t, ws_t, wab, bias2d

    convs = params['convs']
    norms = params['norms']
    kans = params['kans']

    wn_t, ws_t, wab, bias2d = conv_inputs(convs[0])
    h, s, ab = _proj(x, wn_t, ws_t, wab, bias2d)

    for l in range(len(convs) - 1):
        b_att = ab[:, 0]
        a_att = ab[:, 1]
        part, dpart = _sc_aggregate(h, a_att, b_att, src, dst, znd, zn)
        dpart3 = dpart.reshape(NC, N, 1)
        y, s1g, s2g, cntg = _combine_stats(part, dpart3, s, batch2d)
        np_ = norms[l]
        kp = kans[l]
        wn_t, ws_t, wab, bias2d = conv_inputs(convs[l + 1])
        h, s, ab = _norm_kan_proj(
            y, batch2d, s1g, s2g, cntg,
            np_['weight'].reshape(1, D), np_['bias'].reshape(1, D),
            np_['mean_scale'].reshape(1, D),
            kp['weights'].T, kp['bias'].reshape(1, D),
            wn_t, ws_t, wab, bias2d)

    b_att = ab[:, 0]
    a_att = ab[:, 1]
    part, dpart = _sc_aggregate(h, a_att, b_att, src, dst, znd, zn)
    return _final_combine(part, dpart.reshape(NC, N, 1), s)
